# Initial kernel scaffold; baseline (speedup 1.0000x reference)
import jax
import jax.numpy as jnp
from jax.experimental import pallas as pl


def kernel(x, Wq, bq, Wkv, bkv, Wexp):
    raise NotImplementedError("write your pallas kernel here")


# speedup vs baseline: 12.4442x; 12.4442x over previous
import jax
import jax.numpy as jnp
import numpy as np
from jax import lax
from jax.experimental import pallas as pl
from jax.experimental.pallas import tpu as pltpu

H = 12
TOPK = 256
INT_MIN32 = np.int32(-2147483648)


def _kv_body(f_ref, wkv_ref, bkv_ref, out_ref):
    f = f_ref[0]
    out = lax.dot_general(f, wkv_ref[...], (((1,), (1,)), ((), ())),
                          preferred_element_type=jnp.float32)
    out_ref[0] = out + bkv_ref[...]


def _attn_body(tt_ref, wq_ref, bq_ref, e_ref, kv_ref, p_ref, at_ref, T, NF, C,
               HD, scale):
    qrows = []
    for t in range(T):
        qt = lax.dot_general(tt_ref[0, t:t + 1, :], wq_ref[t],
                             (((1,), (1,)), ((), ())),
                             preferred_element_type=jnp.float32)
        qrows.append(qt + bq_ref[t:t + 1, :])
    q = jnp.concatenate(qrows, axis=0)

    e = e_ref[...]
    qp = (e[:, None, :] * q[None, :, :]).reshape(H * T, C)
    k = kv_ref[0, :, :C]
    v = kv_ref[0, :, C:]
    a = lax.dot_general(qp, k, (((1,), (1,)), ((), ())),
                        preferred_element_type=jnp.float32) * scale

    bits = lax.bitcast_convert_type(a, jnp.int32)
    ikey = bits ^ (np.int32(0x7FFFFFFF) & (bits >> 31))
    R = H * T
    tu = jnp.zeros((R, 1), jnp.int32)
    for j in range(31, -1, -1):
        cand = tu | np.int32(1 << j) if j < 31 else tu | INT_MIN32
        thr_s = cand ^ INT_MIN32
        cnt = jnp.sum((ikey >= thr_s).astype(jnp.int32), axis=1, keepdims=True)
        tu = jnp.where(cnt >= TOPK, cand, tu)
    thr = tu ^ INT_MIN32
    mask = ikey >= thr

    rowmax = jnp.max(a, axis=1, keepdims=True)
    ex = jnp.where(mask, jnp.exp(a - rowmax), 0.0)
    denom = jnp.sum(ex, axis=1, keepdims=True)
    p = ex / denom

    p_ref[0] = p.reshape(H, T, NF)

    at3 = lax.dot_general(p, v, (((1,), (0,)), ((), ())),
                          preferred_element_type=jnp.float32)
    at3 = at3.reshape(H, T, C) * e[:, None, :]
    at_ref[0] = jnp.sum(at3, axis=0)


def _feat_body(f_ref, wt_ref, e_ref, wexp_ref, out_ref):
    t = pl.program_id(2)
    w = wt_ref[0, 0]
    wx = lax.dot_general(w, e_ref[...], (((0,), (0,)), ((), ())),
                         preferred_element_type=jnp.float32)
    scaled = f_ref[0] * wx
    contrib = lax.dot_general(scaled, wexp_ref[t], (((1,), (1,)), ((), ())),
                              preferred_element_type=jnp.float32)

    @pl.when(t == 0)
    def _():
        out_ref[0] = contrib

    @pl.when(t > 0)
    def _():
        out_ref[0] += contrib


def _tok_body(at_ref, wexp_ref, out_ref):
    out_ref[0] = lax.dot_general(at_ref[0], wexp_ref[0],
                                 (((1,), (1,)), ((), ())),
                                 preferred_element_type=jnp.float32)


def kernel(x, Wq, bq, Wkv, bkv, Wexp):
    B, N, C = x.shape
    T = Wq.shape[0]
    NF = N - T
    HD = C // H
    scale = HD ** (-0.5)
    C2 = 2 * C

    feature = x[:, T:, :]
    tasktok = x[:, :T, :]
    E = (jnp.arange(C, dtype=jnp.int32)[None, :] // HD
         == jnp.arange(H, dtype=jnp.int32)[:, None]).astype(jnp.float32)

    TN = 512
    NT = NF // TN

    kv = pl.pallas_call(
        _kv_body,
        grid=(B, NT),
        in_specs=[
            pl.BlockSpec((1, TN, C), lambda b, n: (b, n, 0)),
            pl.BlockSpec((C2, C), lambda b, n: (0, 0)),
            pl.BlockSpec((1, C2), lambda b, n: (0, 0)),
        ],
        out_specs=pl.BlockSpec((1, TN, C2), lambda b, n: (b, n, 0)),
        out_shape=jax.ShapeDtypeStruct((B, NF, C2), jnp.float32),
    )(feature, Wkv, bkv.reshape(1, C2))

    attn_fn = lambda tt, wq, bqr, er, kvr, pr, ar: _attn_body(
        tt, wq, bqr, er, kvr, pr, ar, T, NF, C, HD, scale)
    p, at = pl.pallas_call(
        attn_fn,
        grid=(B,),
        in_specs=[
            pl.BlockSpec((1, T, C), lambda b: (b, 0, 0)),
            pl.BlockSpec((T, C, C), lambda b: (0, 0, 0)),
            pl.BlockSpec((T, C), lambda b: (0, 0)),
            pl.BlockSpec((H, C), lambda b: (0, 0)),
            pl.BlockSpec((1, NF, C2), lambda b: (b, 0, 0)),
        ],
        out_specs=[
            pl.BlockSpec((1, H, T, NF), lambda b: (b, 0, 0, 0)),
            pl.BlockSpec((1, T, C), lambda b: (b, 0, 0)),
        ],
        out_shape=[
            jax.ShapeDtypeStruct((B, H, T, NF), jnp.float32),
            jax.ShapeDtypeStruct((B, T, C), jnp.float32),
        ],
    )(tasktok, Wq, bq, E, kv)

    wt = jnp.transpose(p, (0, 2, 1, 3))

    feature_out = pl.pallas_call(
        _feat_body,
        grid=(B, NT, T),
        in_specs=[
            pl.BlockSpec((1, TN, C), lambda b, n, t: (b, n, 0)),
            pl.BlockSpec((1, 1, H, TN), lambda b, n, t: (b, t, 0, n)),
            pl.BlockSpec((H, C), lambda b, n, t: (0, 0)),
            pl.BlockSpec((T, C, C), lambda b, n, t: (0, 0, 0)),
        ],
        out_specs=pl.BlockSpec((1, TN, C), lambda b, n, t: (b, n, 0)),
        out_shape=jax.ShapeDtypeStruct((B, NF, C), jnp.float32),
    )(feature, wt, E, Wexp)

    at_t = jnp.transpose(at, (1, 0, 2))
    tok = pl.pallas_call(
        _tok_body,
        grid=(T,),
        in_specs=[
            pl.BlockSpec((1, B, C), lambda t: (t, 0, 0)),
            pl.BlockSpec((1, C, C), lambda t: (t, 0, 0)),
        ],
        out_specs=pl.BlockSpec((1, B, C), lambda t: (t, 0, 0)),
        out_shape=jax.ShapeDtypeStruct((T, B, C), jnp.float32),
    )(at_t, Wexp)
    token_out = jnp.transpose(tok, (1, 0, 2))

    return jnp.concatenate([token_out, feature_out], axis=1)


# speedup vs baseline: 12.7911x; 1.0279x over previous
import jax
import jax.numpy as jnp
import numpy as np
from jax import lax
from jax.experimental import pallas as pl
from jax.experimental.pallas import tpu as pltpu

H = 12
TOPK = 256
INT_MIN32 = np.int32(-2147483648)


def _kv_body(f_ref, wkv_ref, bkv_ref, out_ref):
    f = f_ref[0]
    out = lax.dot_general(f, wkv_ref[...], (((1,), (1,)), ((), ())),
                          preferred_element_type=jnp.float32)
    out_ref[0] = out + bkv_ref[...]


def _attn_body(tt_ref, wq_ref, bq_ref, e_ref, kv_ref, wexp_ref, p_ref,
               tok_ref, T, NF, C, HD, scale):
    qrows = []
    for t in range(T):
        qt = lax.dot_general(tt_ref[0, t:t + 1, :], wq_ref[t],
                             (((1,), (1,)), ((), ())),
                             preferred_element_type=jnp.float32)
        qrows.append(qt + bq_ref[t:t + 1, :])
    q = jnp.concatenate(qrows, axis=0)

    e = e_ref[...]
    qp = (q[:, None, :] * e[None, :, :]).reshape(T * H, C)
    k = kv_ref[0, :, :C]
    v = kv_ref[0, :, C:]
    a = lax.dot_general(qp, k, (((1,), (1,)), ((), ())),
                        preferred_element_type=jnp.float32) * scale

    bits = lax.bitcast_convert_type(a, jnp.int32)
    ikey = bits ^ (np.int32(0x7FFFFFFF) & (bits >> 31))
    R = H * T
    tu = jnp.zeros((R, 1), jnp.int32)
    for j in range(31, -1, -1):
        cand = tu | np.int32(1 << j) if j < 31 else tu | INT_MIN32
        thr_s = cand ^ INT_MIN32
        cnt = jnp.sum((ikey >= thr_s).astype(jnp.int32), axis=1, keepdims=True)
        tu = jnp.where(cnt >= TOPK, cand, tu)
    thr = tu ^ INT_MIN32
    mask = ikey >= thr

    rowmax = jnp.max(a, axis=1, keepdims=True)
    ex = jnp.where(mask, jnp.exp(a - rowmax), 0.0)
    denom = jnp.sum(ex, axis=1, keepdims=True)
    p = ex / denom

    p_ref[0] = p.reshape(T, H, NF)

    at3 = lax.dot_general(p, v, (((1,), (0,)), ((), ())),
                          preferred_element_type=jnp.float32)
    at = jnp.sum(at3.reshape(T, H, C) * e[None, :, :], axis=1)

    tokrows = []
    for t in range(T):
        tokrows.append(lax.dot_general(at[t:t + 1, :], wexp_ref[t],
                                       (((1,), (1,)), ((), ())),
                                       preferred_element_type=jnp.float32))
    tok_ref[0] = jnp.concatenate(tokrows, axis=0)


def _feat_body(f_ref, wt_ref, e_ref, wexp_ref, out_ref):
    t = pl.program_id(2)
    w = wt_ref[0, 0]
    wx = lax.dot_general(w, e_ref[...], (((0,), (0,)), ((), ())),
                         preferred_element_type=jnp.float32)
    scaled = f_ref[0] * wx
    contrib = lax.dot_general(scaled, wexp_ref[t], (((1,), (1,)), ((), ())),
                              preferred_element_type=jnp.float32)

    @pl.when(t == 0)
    def _():
        out_ref[0] = contrib

    @pl.when(t > 0)
    def _():
        out_ref[0] += contrib


def kernel(x, Wq, bq, Wkv, bkv, Wexp):
    B, N, C = x.shape
    T = Wq.shape[0]
    NF = N - T
    HD = C // H
    scale = HD ** (-0.5)
    C2 = 2 * C

    feature = x[:, T:, :]
    tasktok = x[:, :T, :]
    E = (jnp.arange(C, dtype=jnp.int32)[None, :] // HD
         == jnp.arange(H, dtype=jnp.int32)[:, None]).astype(jnp.float32)

    TN = 512
    NT = NF // TN

    kv = pl.pallas_call(
        _kv_body,
        grid=(B, NT),
        in_specs=[
            pl.BlockSpec((1, TN, C), lambda b, n: (b, n, 0)),
            pl.BlockSpec((C2, C), lambda b, n: (0, 0)),
            pl.BlockSpec((1, C2), lambda b, n: (0, 0)),
        ],
        out_specs=pl.BlockSpec((1, TN, C2), lambda b, n: (b, n, 0)),
        out_shape=jax.ShapeDtypeStruct((B, NF, C2), jnp.float32),
    )(feature, Wkv, bkv.reshape(1, C2))

    attn_fn = lambda tt, wq, bqr, er, kvr, wer, pr, tkr: _attn_body(
        tt, wq, bqr, er, kvr, wer, pr, tkr, T, NF, C, HD, scale)
    wt, token_out = pl.pallas_call(
        attn_fn,
        grid=(B,),
        in_specs=[
            pl.BlockSpec((1, T, C), lambda b: (b, 0, 0)),
            pl.BlockSpec((T, C, C), lambda b: (0, 0, 0)),
            pl.BlockSpec((T, C), lambda b: (0, 0)),
            pl.BlockSpec((H, C), lambda b: (0, 0)),
            pl.BlockSpec((1, NF, C2), lambda b: (b, 0, 0)),
            pl.BlockSpec((T, C, C), lambda b: (0, 0, 0)),
        ],
        out_specs=[
            pl.BlockSpec((1, T, H, NF), lambda b: (b, 0, 0, 0)),
            pl.BlockSpec((1, T, C), lambda b: (b, 0, 0)),
        ],
        out_shape=[
            jax.ShapeDtypeStruct((B, T, H, NF), jnp.float32),
            jax.ShapeDtypeStruct((B, T, C), jnp.float32),
        ],
    )(tasktok, Wq, bq, E, kv, Wexp)

    feature_out = pl.pallas_call(
        _feat_body,
        grid=(B, NT, T),
        in_specs=[
            pl.BlockSpec((1, TN, C), lambda b, n, t: (b, n, 0)),
            pl.BlockSpec((1, 1, H, TN), lambda b, n, t: (b, t, 0, n)),
            pl.BlockSpec((H, C), lambda b, n, t: (0, 0)),
            pl.BlockSpec((T, C, C), lambda b, n, t: (0, 0, 0)),
        ],
        out_specs=pl.BlockSpec((1, TN, C), lambda b, n, t: (b, n, 0)),
        out_shape=jax.ShapeDtypeStruct((B, NF, C), jnp.float32),
    )(feature, wt, E, Wexp)

    return jnp.concatenate([token_out, feature_out], axis=1)


# speedup vs baseline: 24.0775x; 1.8824x over previous
import jax
import jax.numpy as jnp
import numpy as np
from jax import lax
from jax.experimental import pallas as pl
from jax.experimental.pallas import tpu as pltpu

H = 12
TOPK = 256
INT_MIN32 = np.int32(-2147483648)
NEG_BIG = np.float32(-3.0e38)


def _attn_body(x_ref, wq_ref, bq_ref, wkv_ref, bkv_ref, e_ref, g_ref, w2_ref,
               kv_ref, T, N, C, HD, scale):
    TH = T * H
    NFp = N

    xb = x_ref[0]
    kv = lax.dot_general(xb, wkv_ref[...], (((1,), (1,)), ((), ())),
                         preferred_element_type=jnp.float32)
    kv_ref[...] = kv + bkv_ref[...]
    k = kv_ref[:, :C]
    v = kv_ref[:, C:]

    qrows = []
    for t in range(T):
        qt = lax.dot_general(xb[t:t + 1, :], wq_ref[t],
                             (((1,), (1,)), ((), ())),
                             preferred_element_type=jnp.float32)
        qrows.append(qt + bq_ref[t:t + 1, :])
    q = jnp.concatenate(qrows, axis=0)

    e = e_ref[...]
    qp = (q[:, None, :] * e[None, :, :]).reshape(TH, C)

    a = lax.dot_general(qp, k, (((1,), (1,)), ((), ())),
                        preferred_element_type=jnp.float32) * scale
    col = lax.broadcasted_iota(jnp.int32, (TH, NFp), 1)
    a = jnp.where(col < T, NEG_BIG, a)

    bits = lax.bitcast_convert_type(a, jnp.int32)
    ikey = bits ^ (np.int32(0x7FFFFFFF) & (bits >> 31))
    tu = jnp.zeros((TH, 1), jnp.int32)
    for j in range(31, -1, -1):
        cand = tu | np.int32(1 << j) if j < 31 else tu | INT_MIN32
        thr_s = cand ^ INT_MIN32
        cnt = jnp.sum((ikey >= thr_s).astype(jnp.int32), axis=1, keepdims=True)
        tu = jnp.where(cnt >= TOPK, cand, tu)
    thr_s = tu ^ INT_MIN32
    mask = ikey >= thr_s

    rowmax = jnp.max(a, axis=1, keepdims=True)
    ex = jnp.where(mask, jnp.exp(a - rowmax), 0.0)
    denom = jnp.sum(ex, axis=1, keepdims=True)
    p_row = ex / denom

    at3 = lax.dot_general(p_row, v, (((1,), (0,)), ((), ())),
                          preferred_element_type=jnp.float32)
    at = jnp.sum(at3.reshape(T, H, C) * e[None, :, :], axis=1)

    g_ref[0] = xb
    g_ref[0, 0:T, :] = at

    thr_bits = thr_s ^ (np.int32(0x7FFFFFFF) & (thr_s >> 31))
    thr_f = lax.bitcast_convert_type(thr_bits, jnp.float32)
    i36 = (lax.broadcasted_iota(jnp.int32, (TH, TH), 0)
           == lax.broadcasted_iota(jnp.int32, (TH, TH), 1)).astype(jnp.float32)
    stats = jnp.concatenate([thr_f, rowmax, denom], axis=1)
    stats_t = lax.dot_general(stats, i36, (((0,), (0,)), ((), ())),
                              preferred_element_type=jnp.float32)
    thr_r = stats_t[0:1, :]
    max_r = stats_t[1:2, :]
    den_r = stats_t[2:3, :]

    a2 = lax.dot_general(k, qp, (((1,), (1,)), ((), ())),
                         preferred_element_type=jnp.float32) * scale
    row = lax.broadcasted_iota(jnp.int32, (NFp, TH), 0)
    lane = lax.broadcasted_iota(jnp.int32, (NFp, TH), 1)
    a2 = jnp.where(row < T, NEG_BIG, a2)
    p2 = jnp.where(a2 >= thr_r, jnp.exp(a2 - max_r), 0.0) / den_r
    onehot = ((lane // H) == row).astype(jnp.float32)
    w2_ref[0] = jnp.where(row < T, onehot, p2)


def _out_body(g_ref, w2_ref, e_ref, wexp_ref, out_ref, T):
    g = g_ref[0]
    w2 = w2_ref[0]
    acc = None
    for t in range(T):
        wt = w2[:, t * H:(t + 1) * H]
        wx = lax.dot_general(wt, e_ref[...], (((1,), (0,)), ((), ())),
                             preferred_element_type=jnp.float32)
        contrib = lax.dot_general(g * wx, wexp_ref[t],
                                  (((1,), (1,)), ((), ())),
                                  preferred_element_type=jnp.float32)
        acc = contrib if acc is None else acc + contrib
    out_ref[0] = acc


def kernel(x, Wq, bq, Wkv, bkv, Wexp):
    B, N, C = x.shape
    T = Wq.shape[0]
    HD = C // H
    scale = HD ** (-0.5)
    C2 = 2 * C
    TH = T * H

    E = (jnp.arange(C, dtype=jnp.int32)[None, :] // HD
         == jnp.arange(H, dtype=jnp.int32)[:, None]).astype(jnp.float32)

    attn_fn = lambda xr, wq, bqr, wkv, bkvr, er, gr, w2r, kvr: _attn_body(
        xr, wq, bqr, wkv, bkvr, er, gr, w2r, kvr, T, N, C, HD, scale)
    g, w2 = pl.pallas_call(
        attn_fn,
        grid=(B,),
        in_specs=[
            pl.BlockSpec((1, N, C), lambda b: (b, 0, 0)),
            pl.BlockSpec((T, C, C), lambda b: (0, 0, 0)),
            pl.BlockSpec((T, C), lambda b: (0, 0)),
            pl.BlockSpec((C2, C), lambda b: (0, 0)),
            pl.BlockSpec((1, C2), lambda b: (0, 0)),
            pl.BlockSpec((H, C), lambda b: (0, 0)),
        ],
        out_specs=[
            pl.BlockSpec((1, N, C), lambda b: (b, 0, 0)),
            pl.BlockSpec((1, N, TH), lambda b: (b, 0, 0)),
        ],
        out_shape=[
            jax.ShapeDtypeStruct((B, N, C), jnp.float32),
            jax.ShapeDtypeStruct((B, N, TH), jnp.float32),
        ],
        scratch_shapes=[pltpu.VMEM((N, C2), jnp.float32)],
    )(x, Wq, bq, Wkv, bkv.reshape(1, C2), E)

    BS = 296
    NB = -(-N // BS)
    out_fn = lambda gr, w2r, er, wer, outr: _out_body(gr, w2r, er, wer, outr, T)
    out = pl.pallas_call(
        out_fn,
        grid=(B, NB),
        in_specs=[
            pl.BlockSpec((1, BS, C), lambda b, n: (b, n, 0)),
            pl.BlockSpec((1, BS, TH), lambda b, n: (b, n, 0)),
            pl.BlockSpec((H, C), lambda b, n: (0, 0)),
            pl.BlockSpec((T, C, C), lambda b, n: (0, 0, 0)),
        ],
        out_specs=pl.BlockSpec((1, BS, C), lambda b, n: (b, n, 0)),
        out_shape=jax.ShapeDtypeStruct((B, N, C), jnp.float32),
    )(g, w2, E, Wexp)

    return out
